# baseline (speedup 1.0000x reference)
import jax
import jax.numpy as jnp
from jax import lax
from jax.experimental import pallas as pl
from jax.experimental.pallas import tpu as pltpu
from jax.experimental.pallas import tpu_sc as plsc

N = 10000
M = 160000
D = 256

NC = 2
NS = 16
NW = NC * NS

KSLICE = 5
MS = M // KSLICE
EPW = MS // NW
CHUNK = 200
NCHUNK = EPW // CHUNK
BM = 2000
STEPS = MS // BM


def _proj_body(h_ref, wc_ref, o_ref):
    p = lax.dot_general(
        h_ref[...], wc_ref[...],
        dimension_numbers=(((1,), (1,)), ((), ())),
        preferred_element_type=jnp.float32,
    )
    r = p.astype(jnp.bfloat16).astype(jnp.float32)
    u = lax.bitcast_convert_type(r, jnp.uint32)
    s16 = jnp.uint32(16)
    mask = jnp.uint32(0xFFFF0000)
    head = lax.shift_right_logical(u[:, 0:128], s16) | (u[:, 128:256] & mask)
    tail = lax.shift_right_logical(u[:, 256:384], s16) | (u[:, 384:512] & mask)
    o_ref[...] = lax.bitcast_convert_type(
        jnp.concatenate([head, tail], axis=1), jnp.int32)


def _node_table(H, Wc):
    bn = 2000
    return pl.pallas_call(
        _proj_body,
        grid=(N // bn,),
        in_specs=[
            pl.BlockSpec((bn, D), lambda i: (i, 0)),
            pl.BlockSpec((2 * D, D), lambda i: (0, 0)),
        ],
        out_specs=pl.BlockSpec((bn, D), lambda i: (i, 0)),
        out_shape=jax.ShapeDtypeStruct((N, D), jnp.int32),
    )(H, Wc)


def _packed_add(r0, r1):
    @plsc.parallel_loop(0, CHUNK, step=1, unroll=2)
    def row(i):
        for k in range(128 // 16):
            sl = pl.ds(k * 16, 16)
            v0 = lax.bitcast_convert_type(r0[i, sl], jnp.uint32)
            v1 = lax.bitcast_convert_type(r1[i, sl], jnp.uint32)
            s16 = jnp.uint32(16)
            mask = jnp.uint32(0xFFFF0000)
            lo = (lax.bitcast_convert_type(v0 << s16, jnp.float32)
                  + lax.bitcast_convert_type(v1 << s16, jnp.float32))
            hi = (lax.bitcast_convert_type(v0 & mask, jnp.float32)
                  + lax.bitcast_convert_type(v1 & mask, jnp.float32))
            ulo = lax.bitcast_convert_type(lo, jnp.uint32)
            uhi = lax.bitcast_convert_type(hi, jnp.uint32)
            half = jnp.uint32(0x7FFF)
            one = jnp.uint32(1)
            rlo = (ulo + half + ((ulo >> s16) & one)) >> s16
            rhi = (uhi + half + ((uhi >> s16) & one)) & mask
            r0[i, sl] = lax.bitcast_convert_type(rlo | rhi, jnp.int32)


def _sc_gather_body(g_hbm, i0_hbm, i1_hbm, s_hbm,
                    i0_v, i1_v,
                    r0a, r1a, r0b, r1b,
                    gsa, gsb, wsa, wsb):
    wid = lax.axis_index("s") * NC + lax.axis_index("c")
    base = wid * EPW
    pltpu.sync_copy(i0_hbm.at[pl.ds(base, EPW)], i0_v)
    pltpu.sync_copy(i1_hbm.at[pl.ds(base, EPW)], i1_v)

    bufs = ((r0a, r1a, gsa, wsa), (r0b, r1b, gsb, wsb))
    prev_write = [None, None]
    prev_gather = None
    for j in range(NCHUNK):
        bi = j % 2
        r0, r1, gsem, _ = bufs[bi]
        if prev_write[bi] is not None:
            prev_write[bi].wait()
            prev_write[bi] = None
        off = j * CHUNK
        cp0 = pltpu.async_copy(g_hbm.at[i0_v.at[pl.ds(off, CHUNK)]], r0, gsem)
        cp1 = pltpu.async_copy(g_hbm.at[i1_v.at[pl.ds(off, CHUNK)]], r1, gsem)
        if prev_gather is not None:
            g0, g1, pr0, pr1, poff, pbi = prev_gather
            g0.wait()
            g1.wait()
            _packed_add(pr0, pr1)
            prev_write[pbi] = pltpu.async_copy(
                pr0, s_hbm.at[pl.ds(base + poff, CHUNK)], bufs[pbi][3])
        prev_gather = (cp0, cp1, r0, r1, off, bi)

    g0, g1, pr0, pr1, poff, pbi = prev_gather
    g0.wait()
    g1.wait()
    _packed_add(pr0, pr1)
    pltpu.sync_copy(pr0, s_hbm.at[pl.ds(base + poff, CHUNK)])
    for pw in prev_write:
        if pw is not None:
            pw.wait()


def _sc_gather(G2, idx0, idx1):
    mesh = plsc.VectorSubcoreMesh(
        core_axis_name="c", subcore_axis_name="s",
        num_cores=NC, num_subcores=NS,
    )
    fn = pl.kernel(
        _sc_gather_body,
        out_type=jax.ShapeDtypeStruct((MS, 128), jnp.int32),
        mesh=mesh,
        scratch_types=[
            pltpu.VMEM((EPW,), jnp.int32),
            pltpu.VMEM((EPW,), jnp.int32),
            pltpu.VMEM((CHUNK, 128), jnp.int32),
            pltpu.VMEM((CHUNK, 128), jnp.int32),
            pltpu.VMEM((CHUNK, 128), jnp.int32),
            pltpu.VMEM((CHUNK, 128), jnp.int32),
            pltpu.SemaphoreType.DMA,
            pltpu.SemaphoreType.DMA,
            pltpu.SemaphoreType.DMA,
            pltpu.SemaphoreType.DMA,
        ],
    )
    return fn(G2, idx0, idx1)


def _edge_compute(e_ref, s_ref, w2_ref, b_ref, lnw_ref, lnb_ref, o_ref):
    e = e_ref[...]
    f = lax.dot_general(
        e, w2_ref[...],
        dimension_numbers=(((1,), (1,)), ((), ())),
        preferred_element_type=jnp.float32,
    )
    u = lax.bitcast_convert_type(s_ref[...], jnp.uint32)
    s16 = jnp.uint32(16)
    mask = jnp.uint32(0xFFFF0000)
    lo = lax.bitcast_convert_type(lax.shift_left(u, s16), jnp.float32)
    hi = lax.bitcast_convert_type(u & mask, jnp.float32)
    s = jnp.concatenate([lo, hi], axis=1)
    pre = f + s + b_ref[...]
    act = jnp.where(pre >= 0, pre, 0.01 * pre)
    x = act + e
    mu = jnp.mean(x, axis=1, keepdims=True)
    xc = x - mu
    var = jnp.mean(xc * xc, axis=1, keepdims=True)
    inv = lax.rsqrt(var + 1e-5)
    o_ref[...] = xc * inv * lnw_ref[...] + lnb_ref[...]


def _edge_body_first(e_ref, s_ref, w2_ref, b_ref, lnw_ref, lnb_ref, o_ref):
    _edge_compute(e_ref, s_ref, w2_ref, b_ref, lnw_ref, lnb_ref, o_ref)


def _edge_body_cont(prev_ref, e_ref, s_ref, w2_ref, b_ref, lnw_ref,
                    lnb_ref, o_ref):
    del prev_ref
    _edge_compute(e_ref, s_ref, w2_ref, b_ref, lnw_ref, lnb_ref, o_ref)


def _edge_update_slice(out_prev, E, S, W2, b, ln_w, ln_b, k):
    base = STEPS * k
    data_specs = [
        pl.BlockSpec((BM, D), lambda i: (i + base, 0)),
        pl.BlockSpec((BM, 128), lambda i: (i, 0)),
        pl.BlockSpec((D, D), lambda i: (0, 0)),
        pl.BlockSpec((1, D), lambda i: (0, 0)),
        pl.BlockSpec((1, D), lambda i: (0, 0)),
        pl.BlockSpec((1, D), lambda i: (0, 0)),
    ]
    out_spec = pl.BlockSpec((BM, D), lambda i: (i + base, 0))
    out_shape = jax.ShapeDtypeStruct((M, D), jnp.float32)
    if out_prev is None:
        return pl.pallas_call(
            _edge_body_first,
            grid=(STEPS,),
            in_specs=data_specs,
            out_specs=out_spec,
            out_shape=out_shape,
        )(E, S, W2, b, ln_w, ln_b)
    return pl.pallas_call(
        _edge_body_cont,
        grid=(STEPS,),
        in_specs=[pl.BlockSpec(memory_space=pltpu.MemorySpace.HBM)]
        + data_specs,
        out_specs=out_spec,
        out_shape=out_shape,
        input_output_aliases={0: 0},
    )(out_prev, E, S, W2, b, ln_w, ln_b)


@jax.jit
def kernel(H, E, ht, W, b, ln_w, ln_b):
    W1 = W[:, :D]
    W2 = W[:, D:2 * D]
    W3 = W[:, 2 * D:]
    Wc = jnp.concatenate([W1, W3], axis=0)

    G = _node_table(H, Wc)
    G2 = G.reshape(2 * N, 128)

    idx0 = ht[:, 0] * 2
    idx1 = ht[:, 1] * 2 + 1

    b2 = b.reshape(1, D)
    lnw2 = ln_w.reshape(1, D)
    lnb2 = ln_b.reshape(1, D)

    gathered = [
        _sc_gather(G2, idx0[k * MS:(k + 1) * MS], idx1[k * MS:(k + 1) * MS])
        for k in range(KSLICE)
    ]
    out = None
    for k, S in enumerate(gathered):
        out = _edge_update_slice(out, E, S, W2, b2, lnw2, lnb2, k)
    return out


# speedup vs baseline: 1.0505x; 1.0505x over previous
import jax
import jax.numpy as jnp
from jax import lax
from jax.experimental import pallas as pl
from jax.experimental.pallas import tpu as pltpu
from jax.experimental.pallas import tpu_sc as plsc

N = 10000
M = 160000
D = 256

NC = 2
NS = 16
NW = NC * NS

KSLICE = 5
MS = M // KSLICE
EPW = MS // NW
CHUNK = 200
NCHUNK = EPW // CHUNK
BM = 2000
STEPS = MS // BM


def _proj_body(h_ref, wc_ref, o_ref):
    p = lax.dot_general(
        h_ref[...], wc_ref[...],
        dimension_numbers=(((1,), (1,)), ((), ())),
        preferred_element_type=jnp.float32,
    )
    r = p.astype(jnp.bfloat16).astype(jnp.float32)
    u = lax.bitcast_convert_type(r, jnp.uint32)
    s16 = jnp.uint32(16)
    mask = jnp.uint32(0xFFFF0000)
    head = lax.shift_right_logical(u[:, 0:128], s16) | (u[:, 128:256] & mask)
    tail = lax.shift_right_logical(u[:, 256:384], s16) | (u[:, 384:512] & mask)
    o_ref[...] = lax.bitcast_convert_type(
        jnp.concatenate([head, tail], axis=1), jnp.int32)


def _node_table(H, Wc):
    bn = 2000
    return pl.pallas_call(
        _proj_body,
        grid=(N // bn,),
        in_specs=[
            pl.BlockSpec((bn, D), lambda i: (i, 0)),
            pl.BlockSpec((2 * D, D), lambda i: (0, 0)),
        ],
        out_specs=pl.BlockSpec((bn, D), lambda i: (i, 0)),
        out_shape=jax.ShapeDtypeStruct((N, D), jnp.int32),
    )(H, Wc)


def _packed_add(r0, r1):
    def row(i, c):
        for k in range(128 // 16):
            sl = pl.ds(k * 16, 16)
            v0 = lax.bitcast_convert_type(r0[i, sl], jnp.uint32)
            v1 = lax.bitcast_convert_type(r1[i, sl], jnp.uint32)
            s16 = jnp.uint32(16)
            mask = jnp.uint32(0xFFFF0000)
            lo = (lax.bitcast_convert_type(v0 << s16, jnp.float32)
                  + lax.bitcast_convert_type(v1 << s16, jnp.float32))
            hi = (lax.bitcast_convert_type(v0 & mask, jnp.float32)
                  + lax.bitcast_convert_type(v1 & mask, jnp.float32))
            ulo = lax.bitcast_convert_type(lo, jnp.uint32)
            uhi = lax.bitcast_convert_type(hi, jnp.uint32)
            half = jnp.uint32(0x7FFF)
            one = jnp.uint32(1)
            rlo = (ulo + half + ((ulo >> s16) & one)) >> s16
            rhi = (uhi + half + ((uhi >> s16) & one)) & mask
            r0[i, sl] = lax.bitcast_convert_type(rlo | rhi, jnp.int32)
        return c

    lax.fori_loop(0, CHUNK, row, 0, unroll=False)


def _sc_gather_body(g_hbm, i0_hbm, i1_hbm, s_hbm,
                    i0_v, i1_v,
                    r0a, r1a, r0b, r1b,
                    gsa, gsb, wsa, wsb):
    wid = lax.axis_index("s") * NC + lax.axis_index("c")
    base = wid * EPW
    pltpu.sync_copy(i0_hbm.at[pl.ds(base, EPW)], i0_v)
    pltpu.sync_copy(i1_hbm.at[pl.ds(base, EPW)], i1_v)

    bufs = ((r0a, r1a, gsa, wsa), (r0b, r1b, gsb, wsb))
    prev_write = [None, None]
    prev_gather = None
    for j in range(NCHUNK):
        bi = j % 2
        r0, r1, gsem, _ = bufs[bi]
        if prev_write[bi] is not None:
            prev_write[bi].wait()
            prev_write[bi] = None
        off = j * CHUNK
        cp0 = pltpu.async_copy(g_hbm.at[i0_v.at[pl.ds(off, CHUNK)]], r0, gsem)
        cp1 = pltpu.async_copy(g_hbm.at[i1_v.at[pl.ds(off, CHUNK)]], r1, gsem)
        if prev_gather is not None:
            g0, g1, pr0, pr1, poff, pbi = prev_gather
            g0.wait()
            g1.wait()
            _packed_add(pr0, pr1)
            prev_write[pbi] = pltpu.async_copy(
                pr0, s_hbm.at[pl.ds(base + poff, CHUNK)], bufs[pbi][3])
        prev_gather = (cp0, cp1, r0, r1, off, bi)

    g0, g1, pr0, pr1, poff, pbi = prev_gather
    g0.wait()
    g1.wait()
    _packed_add(pr0, pr1)
    pltpu.sync_copy(pr0, s_hbm.at[pl.ds(base + poff, CHUNK)])
    for pw in prev_write:
        if pw is not None:
            pw.wait()


def _sc_gather(G2, idx0, idx1):
    mesh = plsc.VectorSubcoreMesh(
        core_axis_name="c", subcore_axis_name="s",
        num_cores=NC, num_subcores=NS,
    )
    fn = pl.kernel(
        _sc_gather_body,
        out_type=jax.ShapeDtypeStruct((MS, 128), jnp.int32),
        mesh=mesh,
        scratch_types=[
            pltpu.VMEM((EPW,), jnp.int32),
            pltpu.VMEM((EPW,), jnp.int32),
            pltpu.VMEM((CHUNK, 128), jnp.int32),
            pltpu.VMEM((CHUNK, 128), jnp.int32),
            pltpu.VMEM((CHUNK, 128), jnp.int32),
            pltpu.VMEM((CHUNK, 128), jnp.int32),
            pltpu.SemaphoreType.DMA,
            pltpu.SemaphoreType.DMA,
            pltpu.SemaphoreType.DMA,
            pltpu.SemaphoreType.DMA,
        ],
    )
    return fn(G2, idx0, idx1)


def _edge_compute(e_ref, s_ref, w2_ref, b_ref, lnw_ref, lnb_ref, o_ref):
    e = e_ref[...]
    f = lax.dot_general(
        e, w2_ref[...],
        dimension_numbers=(((1,), (1,)), ((), ())),
        preferred_element_type=jnp.float32,
    )
    u = lax.bitcast_convert_type(s_ref[...], jnp.uint32)
    s16 = jnp.uint32(16)
    mask = jnp.uint32(0xFFFF0000)
    lo = lax.bitcast_convert_type(lax.shift_left(u, s16), jnp.float32)
    hi = lax.bitcast_convert_type(u & mask, jnp.float32)
    s = jnp.concatenate([lo, hi], axis=1)
    pre = f + s + b_ref[...]
    act = jnp.where(pre >= 0, pre, 0.01 * pre)
    x = act + e
    mu = jnp.mean(x, axis=1, keepdims=True)
    xc = x - mu
    var = jnp.mean(xc * xc, axis=1, keepdims=True)
    inv = lax.rsqrt(var + 1e-5)
    o_ref[...] = xc * inv * lnw_ref[...] + lnb_ref[...]


def _edge_body_first(e_ref, s_ref, w2_ref, b_ref, lnw_ref, lnb_ref, o_ref):
    _edge_compute(e_ref, s_ref, w2_ref, b_ref, lnw_ref, lnb_ref, o_ref)


def _edge_body_cont(prev_ref, e_ref, s_ref, w2_ref, b_ref, lnw_ref,
                    lnb_ref, o_ref):
    del prev_ref
    _edge_compute(e_ref, s_ref, w2_ref, b_ref, lnw_ref, lnb_ref, o_ref)


def _edge_update_slice(out_prev, E, S, W2, b, ln_w, ln_b, k):
    base = STEPS * k
    data_specs = [
        pl.BlockSpec((BM, D), lambda i: (i + base, 0)),
        pl.BlockSpec((BM, 128), lambda i: (i, 0)),
        pl.BlockSpec((D, D), lambda i: (0, 0)),
        pl.BlockSpec((1, D), lambda i: (0, 0)),
        pl.BlockSpec((1, D), lambda i: (0, 0)),
        pl.BlockSpec((1, D), lambda i: (0, 0)),
    ]
    out_spec = pl.BlockSpec((BM, D), lambda i: (i + base, 0))
    out_shape = jax.ShapeDtypeStruct((M, D), jnp.float32)
    if out_prev is None:
        return pl.pallas_call(
            _edge_body_first,
            grid=(STEPS,),
            in_specs=data_specs,
            out_specs=out_spec,
            out_shape=out_shape,
        )(E, S, W2, b, ln_w, ln_b)
    return pl.pallas_call(
        _edge_body_cont,
        grid=(STEPS,),
        in_specs=[pl.BlockSpec(memory_space=pltpu.MemorySpace.HBM)]
        + data_specs,
        out_specs=out_spec,
        out_shape=out_shape,
        input_output_aliases={0: 0},
    )(out_prev, E, S, W2, b, ln_w, ln_b)


@jax.jit
def kernel(H, E, ht, W, b, ln_w, ln_b):
    W1 = W[:, :D]
    W2 = W[:, D:2 * D]
    W3 = W[:, 2 * D:]
    Wc = jnp.concatenate([W1, W3], axis=0)

    G = _node_table(H, Wc)
    G2 = G.reshape(2 * N, 128)

    idx0 = ht[:, 0] * 2
    idx1 = ht[:, 1] * 2 + 1

    b2 = b.reshape(1, D)
    lnw2 = ln_w.reshape(1, D)
    lnb2 = ln_b.reshape(1, D)

    gathered = [
        _sc_gather(G2, idx0[k * MS:(k + 1) * MS], idx1[k * MS:(k + 1) * MS])
        for k in range(KSLICE)
    ]
    out = None
    for k, S in enumerate(gathered):
        out = _edge_update_slice(out, E, S, W2, b2, lnw2, lnb2, k)
    return out
